# Initial kernel scaffold; baseline (speedup 1.0000x reference)
import functools

import jax
import jax.numpy as jnp
from jax import lax
from jax.experimental import pallas as pl
from jax.experimental.pallas import tpu as pltpu
from jax.experimental.pallas import tpu_sc as plsc

HIDDEN = 128
SEQ = 200
HALF = SEQ // 2
NLANE = 16
NCHUNK = HIDDEN // NLANE


def _rsqrt(x):
    i = plsc.bitcast(x, jnp.int32)
    i = 0x5F3759DF - (i >> 1)
    y = plsc.bitcast(i, jnp.float32)
    half_x = 0.5 * x
    for _ in range(3):
        y = y * (1.5 - half_x * y * y)
    return y


def _sc_embed(ids_hbm, pos_hbm, tok_hbm, gamma_hbm, beta_hbm, out_hbm,
              pos_v, gamma_v, beta_v, idx_v, rows_v, sem):
    nc = 2
    wid = lax.axis_index("s") * nc + lax.axis_index("c")
    batch = out_hbm.shape[0]
    seq_per_w = batch // 32

    pltpu.sync_copy(pos_hbm, pos_v)
    pltpu.sync_copy(gamma_hbm, gamma_v)
    pltpu.sync_copy(beta_hbm, beta_v)

    def seq_body(j, carry):
        b = wid * seq_per_w + j
        pltpu.sync_copy(ids_hbm.at[b], idx_v)
        cp0 = pltpu.async_copy(tok_hbm.at[idx_v.at[0]],
                               rows_v.at[pl.ds(0, HALF)], sem)
        cp1 = pltpu.async_copy(tok_hbm.at[idx_v.at[1]],
                               rows_v.at[pl.ds(HALF, HALF)], sem)
        cp0.wait()
        cp1.wait()

        def row_body(r, c):
            t = [rows_v[r, pl.ds(k * NLANE, NLANE)]
                 + pos_v[r, pl.ds(k * NLANE, NLANE)]
                 for k in range(NCHUNK)]
            s1 = t[0]
            s2 = t[0] * t[0]
            for k in range(1, NCHUNK):
                s1 = s1 + t[k]
                s2 = s2 + t[k] * t[k]
            total = jnp.sum(s1)
            totsq = jnp.sum(s2)
            mean = total * (1.0 / HIDDEN)
            var = totsq * (1.0 / HIDDEN) - mean * mean
            rstd = _rsqrt(jnp.broadcast_to(var + 1e-12, (NLANE,)))
            for k in range(NCHUNK):
                g = gamma_v[pl.ds(k * NLANE, NLANE)]
                bta = beta_v[pl.ds(k * NLANE, NLANE)]
                rows_v[r, pl.ds(k * NLANE, NLANE)] = (
                    (t[k] - mean) * rstd * g + bta)
            return c

        lax.fori_loop(0, SEQ, row_body, 0)
        pltpu.sync_copy(rows_v, out_hbm.at[b])
        return carry

    lax.fori_loop(0, seq_per_w, seq_body, 0)


def kernel(input_ids, token_table, pos_table, gamma, beta):
    batch, seq = input_ids.shape
    ids3 = input_ids.astype(jnp.int32).reshape(batch, 2, HALF)
    pos = pos_table[:seq]

    mesh = plsc.VectorSubcoreMesh(core_axis_name="c", subcore_axis_name="s")
    run = functools.partial(
        pl.kernel,
        out_type=jax.ShapeDtypeStruct((batch, seq, HIDDEN), jnp.float32),
        mesh=mesh,
        scratch_types=[
            pltpu.VMEM((SEQ, HIDDEN), jnp.float32),
            pltpu.VMEM((HIDDEN,), jnp.float32),
            pltpu.VMEM((HIDDEN,), jnp.float32),
            pltpu.VMEM((2, HALF), jnp.int32),
            pltpu.VMEM((SEQ, HIDDEN), jnp.float32),
            pltpu.SemaphoreType.DMA,
        ],
    )(_sc_embed)
    return run(ids3, pos, token_table, gamma, beta)


# speedup vs baseline: 1.5105x; 1.5105x over previous
import functools

import jax
import jax.numpy as jnp
from jax import lax
from jax.experimental import pallas as pl
from jax.experimental.pallas import tpu as pltpu
from jax.experimental.pallas import tpu_sc as plsc

HIDDEN = 128
SEQ = 200
HALF = SEQ // 2
NLANE = 16
NCHUNK = HIDDEN // NLANE


_GATHER_DNUMS = lax.GatherDimensionNumbers(
    offset_dims=(), collapsed_slice_dims=(0,), start_index_map=(0,))


def _shuffle(v, idx):
    return lax.gather(v, idx, _GATHER_DNUMS, (1,),
                      mode=lax.GatherScatterMode.PROMISE_IN_BOUNDS)


def _allsum(v, perms):
    for idx in perms:
        v = v + _shuffle(v, idx)
    return v


def _rsqrt(x):
    i = lax.bitcast_convert_type(x, jnp.int32)
    i = 0x5F3759DF - (i >> 1)
    y = lax.bitcast_convert_type(i, jnp.float32)
    half_x = 0.5 * x
    for _ in range(3):
        y = y * (1.5 - half_x * y * y)
    return y


def _sc_embed(ids_hbm, pos_hbm, tok_hbm, gamma_hbm, beta_hbm, out_hbm,
              pos_v, gamma_v, beta_v, idx_v, rows_v, sem):
    nc = 2
    wid = lax.axis_index("s") * nc + lax.axis_index("c")
    batch = out_hbm.shape[0]
    seq_per_w = batch // 32

    pltpu.sync_copy(pos_hbm, pos_v)
    pltpu.sync_copy(gamma_hbm, gamma_v)
    pltpu.sync_copy(beta_hbm, beta_v)

    def seq_body(j, carry):
        b = wid * seq_per_w + j
        pltpu.sync_copy(ids_hbm.at[b], idx_v)
        cp0 = pltpu.async_copy(tok_hbm.at[idx_v.at[0]],
                               rows_v.at[pl.ds(0, HALF)], sem)
        cp1 = pltpu.async_copy(tok_hbm.at[idx_v.at[1]],
                               rows_v.at[pl.ds(HALF, HALF)], sem)
        cp0.wait()
        cp1.wait()

        lane = lax.iota(jnp.int32, NLANE)
        perms = [(lane ^ s).reshape(NLANE, 1) for s in (8, 4, 2, 1)]

        def row_body(r, c):
            t = [rows_v[r, pl.ds(k * NLANE, NLANE)]
                 + pos_v[r, pl.ds(k * NLANE, NLANE)]
                 for k in range(NCHUNK)]
            s1 = t[0]
            s2 = t[0] * t[0]
            for k in range(1, NCHUNK):
                s1 = s1 + t[k]
                s2 = s2 + t[k] * t[k]
            total = _allsum(s1, perms)
            totsq = _allsum(s2, perms)
            mean = total * (1.0 / HIDDEN)
            var = totsq * (1.0 / HIDDEN) - mean * mean
            rstd = _rsqrt(var + 1e-12)
            for k in range(NCHUNK):
                g = gamma_v[pl.ds(k * NLANE, NLANE)]
                bta = beta_v[pl.ds(k * NLANE, NLANE)]
                rows_v[r, pl.ds(k * NLANE, NLANE)] = (
                    (t[k] - mean) * rstd * g + bta)
            return c

        lax.fori_loop(0, SEQ, row_body, 0)
        pltpu.sync_copy(rows_v, out_hbm.at[b])
        return carry

    lax.fori_loop(0, seq_per_w, seq_body, 0)


def kernel(input_ids, token_table, pos_table, gamma, beta):
    batch, seq = input_ids.shape
    ids3 = input_ids.astype(jnp.int32).reshape(batch, 2, HALF)
    pos = pos_table[:seq]

    mesh = plsc.VectorSubcoreMesh(core_axis_name="c", subcore_axis_name="s")
    run = functools.partial(
        pl.kernel,
        out_type=jax.ShapeDtypeStruct((batch, seq, HIDDEN), jnp.float32),
        mesh=mesh,
        scratch_types=[
            pltpu.VMEM((SEQ, HIDDEN), jnp.float32),
            pltpu.VMEM((HIDDEN,), jnp.float32),
            pltpu.VMEM((HIDDEN,), jnp.float32),
            pltpu.VMEM((2, HALF), jnp.int32),
            pltpu.VMEM((SEQ, HIDDEN), jnp.float32),
            pltpu.SemaphoreType.DMA,
        ],
    )(_sc_embed)
    return run(ids3, pos, token_table, gamma, beta)


# speedup vs baseline: 2.2398x; 1.4828x over previous
import functools

import jax
import jax.numpy as jnp
from jax import lax
from jax.experimental import pallas as pl
from jax.experimental.pallas import tpu as pltpu
from jax.experimental.pallas import tpu_sc as plsc

HIDDEN = 128
SEQ = 200
CHUNK = 100
NLANE = 16
NCHUNK = HIDDEN // NLANE
NWORK = 32
NBUF = 4

_GATHER_DNUMS = lax.GatherDimensionNumbers(
    offset_dims=(), collapsed_slice_dims=(0,), start_index_map=(0,))


def _shuffle(v, idx):
    return lax.gather(v, idx, _GATHER_DNUMS, (1,),
                      mode=lax.GatherScatterMode.PROMISE_IN_BOUNDS)


def _allsum(v, perms):
    for idx in perms:
        v = v + _shuffle(v, idx)
    return v


def _rsqrt(x):
    i = lax.bitcast_convert_type(x, jnp.int32)
    i = 0x5F3759DF - (i >> 1)
    y = lax.bitcast_convert_type(i, jnp.float32)
    half_x = 0.5 * x
    for _ in range(2):
        y = y * (1.5 - half_x * y * y)
    return y


def _sc_embed(ids_hbm, pos_hbm, tok_hbm, gamma_hbm, beta_hbm, out_hbm,
              pos_v, gamma_v, beta_v, idx_v, bufs, gsems, osems):
    nc = 2
    wid = lax.axis_index("s") * nc + lax.axis_index("c")
    nchunks = ids_hbm.shape[0]
    per_w = nchunks // NWORK
    base = wid * per_w

    pltpu.sync_copy(pos_hbm, pos_v)
    pltpu.sync_copy(gamma_hbm, gamma_v)
    pltpu.sync_copy(beta_hbm, beta_v)
    pltpu.sync_copy(ids_hbm.at[pl.ds(base, per_w)], idx_v)

    def drain(sem, buf):
        pltpu.make_async_copy(out_hbm.at[0], buf, sem).wait()

    def gather(j, s):
        pltpu.async_copy(tok_hbm.at[idx_v.at[j]], bufs[s], gsems[s])

    lane = lax.iota(jnp.int32, NLANE)
    perms = [(lane ^ sh).reshape(NLANE, 1) for sh in (8, 4, 2, 1)]
    g = [gamma_v[pl.ds(k * NLANE, NLANE)] for k in range(NCHUNK)]
    bt = [beta_v[pl.ds(k * NLANE, NLANE)] for k in range(NCHUNK)]

    def compute(buf, pos_off):
        def row_body(r, c):
            t = [buf[r, pl.ds(k * NLANE, NLANE)]
                 + pos_v[pos_off + r, pl.ds(k * NLANE, NLANE)]
                 for k in range(NCHUNK)]
            s1 = t[0]
            s2 = t[0] * t[0]
            for k in range(1, NCHUNK):
                s1 = s1 + t[k]
                s2 = s2 + t[k] * t[k]
            total = _allsum(s1, perms)
            totsq = _allsum(s2, perms)
            mean = total * (1.0 / HIDDEN)
            var = totsq * (1.0 / HIDDEN) - mean * mean
            rstd = _rsqrt(var + 1e-12)
            for k in range(NCHUNK):
                buf[r, pl.ds(k * NLANE, NLANE)] = (
                    (t[k] - mean) * rstd * g[k] + bt[k])
            return c
        lax.fori_loop(0, CHUNK, row_body, 0)

    for s in range(NBUF - 1):
        gather(s, s)

    def iter_body(it, carry):
        for s in range(NBUF):
            q = it * NBUF + s
            s_next = (s + NBUF - 1) % NBUF

            if s == 0:
                @pl.when(it > 0)
                def _():
                    drain(osems[s_next], bufs[s_next])
                gather(q + NBUF - 1, s_next)
            else:
                @pl.when(it < (per_w // NBUF) - 1)
                def _():
                    drain(osems[s_next], bufs[s_next])
                    gather(q + NBUF - 1, s_next)

            drain(gsems[s], bufs[s])
            compute(bufs[s], (s % 2) * CHUNK)
            pltpu.async_copy(bufs[s], out_hbm.at[base + q], osems[s])
        return carry

    lax.fori_loop(0, per_w // NBUF, iter_body, 0)
    for s in range(NBUF):
        drain(osems[s], bufs[s])


def kernel(input_ids, token_table, pos_table, gamma, beta):
    batch, seq = input_ids.shape
    nchunks = batch * seq // CHUNK
    ids2 = input_ids.astype(jnp.int32).reshape(nchunks, CHUNK)
    pos = pos_table[:seq]

    mesh = plsc.VectorSubcoreMesh(core_axis_name="c", subcore_axis_name="s")
    run = functools.partial(
        pl.kernel,
        out_type=jax.ShapeDtypeStruct((nchunks, CHUNK, HIDDEN), jnp.float32),
        mesh=mesh,
        scratch_types=[
            pltpu.VMEM((SEQ, HIDDEN), jnp.float32),
            pltpu.VMEM((HIDDEN,), jnp.float32),
            pltpu.VMEM((HIDDEN,), jnp.float32),
            pltpu.VMEM((nchunks // NWORK, CHUNK), jnp.int32),
            [pltpu.VMEM((CHUNK, HIDDEN), jnp.float32)] * NBUF,
            [pltpu.SemaphoreType.DMA] * NBUF,
            [pltpu.SemaphoreType.DMA] * NBUF,
        ],
    )(_sc_embed)
    out = run(ids2, pos, token_table, gamma, beta)
    return out.reshape(batch, seq, HIDDEN)


# speedup vs baseline: 3.1370x; 1.4006x over previous
import functools

import jax
import jax.numpy as jnp
from jax import lax
from jax.experimental import pallas as pl
from jax.experimental.pallas import tpu as pltpu
from jax.experimental.pallas import tpu_sc as plsc

HIDDEN = 128
SEQ = 200
CHUNK = 100
NLANE = 16
NCHUNK = HIDDEN // NLANE
NWORK = 32
NBUF = 4

_GATHER_DNUMS = lax.GatherDimensionNumbers(
    offset_dims=(), collapsed_slice_dims=(0,), start_index_map=(0,))


def _shuffle(v, idx):
    return lax.gather(v, idx, _GATHER_DNUMS, (1,),
                      mode=lax.GatherScatterMode.PROMISE_IN_BOUNDS)


def _allsum(v, perms):
    for idx in perms:
        v = v + _shuffle(v, idx)
    return v


def _rsqrt(x):
    i = lax.bitcast_convert_type(x, jnp.int32)
    i = 0x5F3759DF - (i >> 1)
    y = lax.bitcast_convert_type(i, jnp.float32)
    half_x = 0.5 * x
    for _ in range(2):
        y = y * (1.5 - half_x * y * y)
    return y


def _sc_embed(ids_hbm, pos_hbm, tok_hbm, gamma_hbm, beta_hbm, out_hbm,
              pos_v, gamma_v, beta_v, idx_v, bufs, gsems, osems):
    nc = 2
    wid = lax.axis_index("s") * nc + lax.axis_index("c")
    nchunks = ids_hbm.shape[0]
    per_w = nchunks // NWORK
    base = wid * per_w

    pltpu.sync_copy(pos_hbm, pos_v)
    pltpu.sync_copy(gamma_hbm, gamma_v)
    pltpu.sync_copy(beta_hbm, beta_v)
    pltpu.sync_copy(ids_hbm.at[pl.ds(base, per_w)], idx_v)

    def drain(sem, buf):
        pltpu.make_async_copy(out_hbm.at[0], buf, sem).wait()

    def gather(j, s):
        pltpu.async_copy(tok_hbm.at[idx_v.at[j]], bufs[s], gsems[s])

    lane = lax.iota(jnp.int32, NLANE)
    perms = [(lane ^ sh).reshape(NLANE, 1) for sh in (8, 4, 2, 1)]
    g = [gamma_v[pl.ds(k * NLANE, NLANE)] for k in range(NCHUNK)]
    bt = [beta_v[pl.ds(k * NLANE, NLANE)] for k in range(NCHUNK)]

    def compute(buf, pos_off):
        def one_row(r):
            t = [buf[r, pl.ds(k * NLANE, NLANE)]
                 + pos_v[pos_off + r, pl.ds(k * NLANE, NLANE)]
                 for k in range(NCHUNK)]
            s1 = t[0]
            s2 = t[0] * t[0]
            for k in range(1, NCHUNK):
                s1 = s1 + t[k]
                s2 = s2 + t[k] * t[k]
            total = _allsum(s1, perms)
            totsq = _allsum(s2, perms)
            mean = total * (1.0 / HIDDEN)
            var = totsq * (1.0 / HIDDEN) - mean * mean
            rstd = _rsqrt(var + 1e-12)
            for k in range(NCHUNK):
                buf[r, pl.ds(k * NLANE, NLANE)] = (
                    (t[k] - mean) * rstd * g[k] + bt[k])

        def row_body(r, c):
            for u in range(4):
                one_row(r * 4 + u)
            return c
        lax.fori_loop(0, CHUNK // 4, row_body, 0)

    for s in range(NBUF - 1):
        gather(s, s)

    def iter_body(it, carry):
        for s in range(NBUF):
            q = it * NBUF + s
            s_next = (s + NBUF - 1) % NBUF

            if s == 0:
                @pl.when(it > 0)
                def _():
                    drain(osems[s_next], bufs[s_next])
                gather(q + NBUF - 1, s_next)
            else:
                @pl.when(it < (per_w // NBUF) - 1)
                def _():
                    drain(osems[s_next], bufs[s_next])
                    gather(q + NBUF - 1, s_next)

            drain(gsems[s], bufs[s])
            compute(bufs[s], (s % 2) * CHUNK)
            pltpu.async_copy(bufs[s], out_hbm.at[base + q], osems[s])
        return carry

    lax.fori_loop(0, per_w // NBUF, iter_body, 0)
    for s in range(NBUF):
        drain(osems[s], bufs[s])


def kernel(input_ids, token_table, pos_table, gamma, beta):
    batch, seq = input_ids.shape
    nchunks = batch * seq // CHUNK
    ids2 = input_ids.astype(jnp.int32).reshape(nchunks, CHUNK)
    pos = pos_table[:seq]

    mesh = plsc.VectorSubcoreMesh(core_axis_name="c", subcore_axis_name="s")
    run = functools.partial(
        pl.kernel,
        out_type=jax.ShapeDtypeStruct((nchunks, CHUNK, HIDDEN), jnp.float32),
        mesh=mesh,
        scratch_types=[
            pltpu.VMEM((SEQ, HIDDEN), jnp.float32),
            pltpu.VMEM((HIDDEN,), jnp.float32),
            pltpu.VMEM((HIDDEN,), jnp.float32),
            pltpu.VMEM((nchunks // NWORK, CHUNK), jnp.int32),
            [pltpu.VMEM((CHUNK, HIDDEN), jnp.float32)] * NBUF,
            [pltpu.SemaphoreType.DMA] * NBUF,
            [pltpu.SemaphoreType.DMA] * NBUF,
        ],
    )(_sc_embed)
    out = run(ids2, pos, token_table, gamma, beta)
    return out.reshape(batch, seq, HIDDEN)
